# baseline (speedup 1.0000x reference)
import functools

import jax
import jax.numpy as jnp
from jax import lax
from jax.experimental import pallas as pl
from jax.experimental.pallas import tpu as pltpu
from jax.experimental.pallas import tpu_sc as plsc

VOCAB = 100000
DIM = 64
N_CTX = 2
DOUT = N_CTX * DIM
B = 4096
L = 50
LP = L // 2
N = B * L

NC = 2
NS = 16
NW = NC * NS
PER_W = N // NW
CHUNK = 128
NCH = PER_W // CHUNK
BPW = B // NW


_sc_mesh = plsc.VectorSubcoreMesh(core_axis_name="c", subcore_axis_name="s")


def _make_gather(nplanes, p_off):
    n_out = nplanes * 2 * B
    nch = n_out // NW // CHUNK

    @functools.partial(
        pl.kernel,
        out_type=jax.ShapeDtypeStruct((n_out, DIM), jnp.float32),
        mesh=_sc_mesh,
        compiler_params=pltpu.CompilerParams(
            use_tc_tiling_on_sc=False, needs_layout_passes=False),
        scratch_types=[
            pltpu.VMEM((BPW * 128,), jnp.int32),
            pltpu.VMEM((CHUNK,), jnp.int32),
            pltpu.VMEM((CHUNK,), jnp.int32),
            pltpu.VMEM((CHUNK, DIM), jnp.float32),
            pltpu.VMEM((CHUNK, DIM), jnp.float32),
            pltpu.SemaphoreType.DMA,
            pltpu.SemaphoreType.DMA,
        ],
    )
    def _sc_gather(tab_hbm, idx_hbm, out_hbm,
                   slab_v, idxc0, idxc1, buf0, buf1, sem0, sem1):
        wid = lax.axis_index("s") * NC + lax.axis_index("c")

        pltpu.sync_copy(idx_hbm.at[pl.ds(wid * BPW * 128, BPW * 128)], slab_v)

        lam = lax.iota(jnp.int32, 16)
        off = (lam >> 1) * 128 + (lam & 1)

        idxcs = (idxc0, idxc1)
        bufs = (buf0, buf1)
        sems = (sem0, sem1)

        def _build(c, slot):
            p = c >> 1
            h = c & 1
            dst = idxcs[slot]
            base0 = h * (CHUNK // 2) * 128 + 2 * (p + p_off)
            for g in range(CHUNK // 16):
                sv = off + (base0 + (g * 8) * L)
                v = plsc.load_gather(slab_v, [sv])
                dst[pl.ds(g * 16, 16)] = v

        def _out_base(c):
            p = c >> 1
            h = c & 1
            return p * (2 * B) + wid * (2 * BPW) + h * CHUNK

        def _start(slot):
            pltpu.make_async_copy(
                tab_hbm.at[idxcs[slot]], bufs[slot], sems[slot]).start()

        def _finish(c, slot):
            pltpu.make_async_copy(
                tab_hbm.at[idxcs[slot]], bufs[slot], sems[slot]).wait()
            pltpu.sync_copy(bufs[slot],
                            out_hbm.at[pl.ds(_out_base(c), CHUNK)])

        _build(0, 0)
        _start(0)

        def _body(t, carry):
            c0 = 2 * t
            _build(c0 + 1, 1)
            _start(1)
            _finish(c0, 0)

            @pl.when(c0 + 2 < nch)
            def _():
                _build(c0 + 2, 0)
                _start(0)

            _finish(c0 + 1, 1)
            return carry

        lax.fori_loop(0, nch // 2, _body, 0)

    return _sc_gather


_PSPLIT = 12
_gather_a = _make_gather(_PSPLIT, 0)
_gather_b = _make_gather(LP - _PSPLIT, _PSPLIT)


_BB = 2048
_NQ = 12


def _quad_body(e_ref, w_ref, o_ref):
    eq = jnp.concatenate([e_ref[0], e_ref[1]], axis=1)
    res = lax.dot_general(
        w_ref[...], eq,
        dimension_numbers=(((1,), (1,)), ((), ())),
        preferred_element_type=jnp.float32)
    for r in range(4):
        o_ref[r] = res[DOUT * r:DOUT * (r + 1)]


def _quad_body_aliased(o_prev_ref, e_ref, w_ref, o_ref):
    del o_prev_ref
    _quad_body(e_ref, w_ref, o_ref)


def _project_quads_a(e2, w4):
    return pl.pallas_call(
        _quad_body,
        grid=(_PSPLIT // 2, B // _BB),
        in_specs=[
            pl.BlockSpec((2, _BB, 2 * DIM), lambda g, b: (g, b, 0)),
            pl.BlockSpec((4 * DOUT, 4 * DIM), lambda g, b: (0, 0)),
        ],
        out_specs=pl.BlockSpec((4, DOUT, _BB), lambda g, b: (g, 0, b)),
        out_shape=jax.ShapeDtypeStruct((L, DOUT, B), jnp.float32),
    )(e2, w4)


def _project_quads_b(out_prev, e2, w4):
    goff = _PSPLIT // 2
    return pl.pallas_call(
        _quad_body_aliased,
        grid=(_PSPLIT // 2, B // _BB),
        in_specs=[
            pl.BlockSpec(memory_space=pl.ANY),
            pl.BlockSpec((2, _BB, 2 * DIM), lambda g, b: (g, b, 0)),
            pl.BlockSpec((4 * DOUT, 4 * DIM), lambda g, b: (0, 0)),
        ],
        out_specs=pl.BlockSpec((4, DOUT, _BB), lambda g, b: (g + goff, 0, b)),
        out_shape=jax.ShapeDtypeStruct((L, DOUT, B), jnp.float32),
        input_output_aliases={0: 0},
    )(out_prev, e2, w4)


def _pair_body(o_prev_ref, e_ref, w_ref, o_ref):
    del o_prev_ref
    res = lax.dot_general(
        w_ref[...], e_ref[0],
        dimension_numbers=(((1,), (1,)), ((), ())),
        preferred_element_type=jnp.float32)
    o_ref[0] = res[:DOUT]
    o_ref[1] = res[DOUT:]


def _project_last_pair(out_prev, e2, w2):
    nb = LP - _PSPLIT
    return pl.pallas_call(
        _pair_body,
        grid=(B // _BB,),
        in_specs=[
            pl.BlockSpec(memory_space=pl.ANY),
            pl.BlockSpec((1, _BB, 2 * DIM), lambda b: (nb - 1, b, 0)),
            pl.BlockSpec((2 * DOUT, 2 * DIM), lambda b: (0, 0)),
        ],
        out_specs=pl.BlockSpec((2, DOUT, _BB), lambda b: (LP - 1, 0, b)),
        out_shape=jax.ShapeDtypeStruct((L, DOUT, B), jnp.float32),
        input_output_aliases={0: 0},
    )(out_prev, e2, w2)


def kernel(x, table, linear_layers_in):
    idx = jnp.pad(x.astype(jnp.int32), ((0, 0), (0, 128 - L))).reshape(B * 128)
    emb_a = _gather_a(table, idx)
    emb_b = _gather_b(table, idx)
    e2a = emb_a.reshape(_PSPLIT, B, 2 * DIM)
    e2b = emb_b.reshape(LP - _PSPLIT, B, 2 * DIM)
    w_t = jnp.transpose(linear_layers_in, (0, 2, 1)).reshape(DOUT, DIM)
    w4 = jnp.einsum('gh,ce->gche', jnp.eye(4, dtype=jnp.float32),
                    w_t).reshape(4 * DOUT, 4 * DIM)
    w2 = w4[:2 * DOUT, :2 * DIM]
    out_q = _project_quads_a(e2a, w4)
    out_q = _project_quads_b(out_q, e2b, w4)
    out_phys = _project_last_pair(out_q, e2b, w2)
    return out_phys.reshape(L, N_CTX, DIM, B).transpose(3, 0, 1, 2)


# speedup vs baseline: 14.5066x; 14.5066x over previous
import functools

import jax
import jax.numpy as jnp
from jax import lax
from jax.experimental import pallas as pl
from jax.experimental.pallas import tpu as pltpu
from jax.experimental.pallas import tpu_sc as plsc

VOCAB = 100000
DIM = 64
N_CTX = 2
DOUT = N_CTX * DIM
B = 4096
L = 50
LP = L // 2
N = B * L

NC = 2
NS = 16
NW = NC * NS
PER_W = N // NW
CHUNK = 128
NCH = PER_W // CHUNK
BPW = B // NW


_sc_mesh = plsc.VectorSubcoreMesh(core_axis_name="c", subcore_axis_name="s")


def _make_gather(nplanes, p_off):
    n_out = nplanes * 2 * B
    nch = n_out // NW // CHUNK

    @functools.partial(
        pl.kernel,
        out_type=jax.ShapeDtypeStruct((n_out, DIM), jnp.float32),
        mesh=_sc_mesh,
        compiler_params=pltpu.CompilerParams(
            use_tc_tiling_on_sc=False, needs_layout_passes=False),
        scratch_types=[
            pltpu.VMEM((BPW * 128,), jnp.int32),
            pltpu.VMEM((CHUNK,), jnp.int32),
            pltpu.VMEM((CHUNK,), jnp.int32),
            pltpu.VMEM((CHUNK, DIM), jnp.float32),
            pltpu.VMEM((CHUNK, DIM), jnp.float32),
            pltpu.SemaphoreType.DMA,
            pltpu.SemaphoreType.DMA,
        ],
    )
    def _sc_gather(tab_hbm, idx_hbm, out_hbm,
                   slab_v, idxc0, idxc1, buf0, buf1, sem0, sem1):
        wid = lax.axis_index("s") * NC + lax.axis_index("c")

        pltpu.sync_copy(idx_hbm.at[pl.ds(wid * BPW * 128, BPW * 128)], slab_v)

        lam = lax.iota(jnp.int32, 16)
        off = (lam >> 1) * 128 + (lam & 1)

        idxcs = (idxc0, idxc1)
        bufs = (buf0, buf1)
        sems = (sem0, sem1)

        def _build(c, slot):
            p = c >> 1
            h = c & 1
            dst = idxcs[slot]
            base0 = h * (CHUNK // 2) * 128 + 2 * (p + p_off)
            for g in range(CHUNK // 16):
                sv = off + (base0 + (g * 8) * 128)
                v = plsc.load_gather(slab_v, [sv])
                dst[pl.ds(g * 16, 16)] = v

        def _out_base(c):
            p = c >> 1
            h = c & 1
            return p * (2 * B) + wid * (2 * BPW) + h * CHUNK

        def _start(slot):
            pltpu.make_async_copy(
                tab_hbm.at[idxcs[slot]], bufs[slot], sems[slot]).start()

        def _finish(c, slot):
            pltpu.make_async_copy(
                tab_hbm.at[idxcs[slot]], bufs[slot], sems[slot]).wait()
            pltpu.sync_copy(bufs[slot],
                            out_hbm.at[pl.ds(_out_base(c), CHUNK)])

        _build(0, 0)
        _start(0)

        def _body(t, carry):
            c0 = 2 * t
            _build(c0 + 1, 1)
            _start(1)
            _finish(c0, 0)

            @pl.when(c0 + 2 < nch)
            def _():
                _build(c0 + 2, 0)
                _start(0)

            _finish(c0 + 1, 1)
            return carry

        lax.fori_loop(0, nch // 2, _body, 0)

    return _sc_gather


_PSPLIT = 12
_gather_a = _make_gather(_PSPLIT, 0)
_gather_b = _make_gather(LP - _PSPLIT, _PSPLIT)


_BB = 2048
_NQ = 12


def _quad_body(e_ref, w_ref, o_ref):
    eq = jnp.concatenate([e_ref[0], e_ref[1]], axis=1)
    res = lax.dot_general(
        w_ref[...], eq,
        dimension_numbers=(((1,), (1,)), ((), ())),
        preferred_element_type=jnp.float32)
    for r in range(4):
        o_ref[r] = res[DOUT * r:DOUT * (r + 1)]


def _quad_body_aliased(o_prev_ref, e_ref, w_ref, o_ref):
    del o_prev_ref
    _quad_body(e_ref, w_ref, o_ref)


def _project_quads_a(e2, w4):
    return pl.pallas_call(
        _quad_body,
        grid=(_PSPLIT // 2, B // _BB),
        in_specs=[
            pl.BlockSpec((2, _BB, 2 * DIM), lambda g, b: (g, b, 0)),
            pl.BlockSpec((4 * DOUT, 4 * DIM), lambda g, b: (0, 0)),
        ],
        out_specs=pl.BlockSpec((4, DOUT, _BB), lambda g, b: (g, 0, b)),
        out_shape=jax.ShapeDtypeStruct((L, DOUT, B), jnp.float32),
    )(e2, w4)


def _project_quads_b(out_prev, e2, w4):
    goff = _PSPLIT // 2
    return pl.pallas_call(
        _quad_body_aliased,
        grid=(_PSPLIT // 2, B // _BB),
        in_specs=[
            pl.BlockSpec(memory_space=pl.ANY),
            pl.BlockSpec((2, _BB, 2 * DIM), lambda g, b: (g, b, 0)),
            pl.BlockSpec((4 * DOUT, 4 * DIM), lambda g, b: (0, 0)),
        ],
        out_specs=pl.BlockSpec((4, DOUT, _BB), lambda g, b: (g + goff, 0, b)),
        out_shape=jax.ShapeDtypeStruct((L, DOUT, B), jnp.float32),
        input_output_aliases={0: 0},
    )(out_prev, e2, w4)


def _pair_body(o_prev_ref, e_ref, w_ref, o_ref):
    del o_prev_ref
    res = lax.dot_general(
        w_ref[...], e_ref[0],
        dimension_numbers=(((1,), (1,)), ((), ())),
        preferred_element_type=jnp.float32)
    o_ref[0] = res[:DOUT]
    o_ref[1] = res[DOUT:]


def _project_last_pair(out_prev, e2, w2):
    nb = LP - _PSPLIT
    return pl.pallas_call(
        _pair_body,
        grid=(B // _BB,),
        in_specs=[
            pl.BlockSpec(memory_space=pl.ANY),
            pl.BlockSpec((1, _BB, 2 * DIM), lambda b: (nb - 1, b, 0)),
            pl.BlockSpec((2 * DOUT, 2 * DIM), lambda b: (0, 0)),
        ],
        out_specs=pl.BlockSpec((2, DOUT, _BB), lambda b: (LP - 1, 0, b)),
        out_shape=jax.ShapeDtypeStruct((L, DOUT, B), jnp.float32),
        input_output_aliases={0: 0},
    )(out_prev, e2, w2)


def kernel(x, table, linear_layers_in):
    idx = jnp.pad(x.astype(jnp.int32), ((0, 0), (0, 128 - L))).reshape(B * 128)
    emb_a = _gather_a(table, idx)
    emb_b = _gather_b(table, idx)
    e2a = emb_a.reshape(_PSPLIT, B, 2 * DIM)
    e2b = emb_b.reshape(LP - _PSPLIT, B, 2 * DIM)
    w_t = jnp.transpose(linear_layers_in, (0, 2, 1)).reshape(DOUT, DIM)
    w4 = jnp.einsum('gh,ce->gche', jnp.eye(4, dtype=jnp.float32),
                    w_t).reshape(4 * DOUT, 4 * DIM)
    w2 = w4[:2 * DOUT, :2 * DIM]
    out_q = _project_quads_a(e2a, w4)
    out_q = _project_quads_b(out_q, e2b, w4)
    out_phys = _project_last_pair(out_q, e2b, w2)
    return out_phys.reshape(L, N_CTX, DIM, B).transpose(3, 0, 1, 2)


# speedup vs baseline: 14.6761x; 1.0117x over previous
import functools

import jax
import jax.numpy as jnp
from jax import lax
from jax.experimental import pallas as pl
from jax.experimental.pallas import tpu as pltpu
from jax.experimental.pallas import tpu_sc as plsc

VOCAB = 100000
DIM = 64
N_CTX = 2
DOUT = N_CTX * DIM
B = 4096
L = 50
LP = L // 2
N = B * L

NC = 2
NS = 16
NW = NC * NS
PER_W = N // NW
CHUNK = 128
NCH = PER_W // CHUNK
BPW = B // NW


_sc_mesh = plsc.VectorSubcoreMesh(core_axis_name="c", subcore_axis_name="s")


def _make_gather(nplanes, p_off):
    n_out = nplanes * 2 * B
    nch = n_out // NW // CHUNK

    @functools.partial(
        pl.kernel,
        out_type=jax.ShapeDtypeStruct((n_out, DIM), jnp.float32),
        mesh=_sc_mesh,
        compiler_params=pltpu.CompilerParams(
            use_tc_tiling_on_sc=False, needs_layout_passes=False),
        scratch_types=[
            pltpu.VMEM((BPW * 128,), jnp.int32),
            pltpu.VMEM((CHUNK,), jnp.int32),
            pltpu.VMEM((CHUNK,), jnp.int32),
            pltpu.VMEM((CHUNK, DIM), jnp.float32),
            pltpu.VMEM((CHUNK, DIM), jnp.float32),
            pltpu.SemaphoreType.DMA,
            pltpu.SemaphoreType.DMA,
        ],
    )
    def _sc_gather(tab_hbm, idx_hbm, out_hbm,
                   slab_v, idxc0, idxc1, buf0, buf1, sem0, sem1):
        wid = lax.axis_index("s") * NC + lax.axis_index("c")

        pltpu.sync_copy(idx_hbm.at[pl.ds(wid * BPW * 128, BPW * 128)], slab_v)

        lam = lax.iota(jnp.int32, 16)
        off = (lam >> 1) * 128 + (lam & 1)

        idxcs = (idxc0, idxc1)
        bufs = (buf0, buf1)
        sems = (sem0, sem1)

        def _build(c, slot):
            p = c >> 1
            h = c & 1
            dst = idxcs[slot]
            base0 = h * (CHUNK // 2) * 128 + 2 * (p + p_off)
            for g in range(CHUNK // 16):
                sv = off + (base0 + (g * 8) * 128)
                v = plsc.load_gather(slab_v, [sv])
                dst[pl.ds(g * 16, 16)] = v

        def _out_base(c):
            p = c >> 1
            h = c & 1
            return p * (2 * B) + wid * (2 * BPW) + h * CHUNK

        def _start(slot):
            pltpu.make_async_copy(
                tab_hbm.at[idxcs[slot]], bufs[slot], sems[slot]).start()

        def _finish(c, slot):
            pltpu.make_async_copy(
                tab_hbm.at[idxcs[slot]], bufs[slot], sems[slot]).wait()
            pltpu.sync_copy(bufs[slot],
                            out_hbm.at[pl.ds(_out_base(c), CHUNK)])

        _build(0, 0)
        _start(0)

        def _body(t, carry):
            c0 = 2 * t
            _build(c0 + 1, 1)
            _start(1)
            _finish(c0, 0)

            @pl.when(c0 + 2 < nch)
            def _():
                _build(c0 + 2, 0)
                _start(0)

            _finish(c0 + 1, 1)
            return carry

        lax.fori_loop(0, nch // 2, _body, 0)

    return _sc_gather


_PSPLIT = 12
_gather_a = _make_gather(_PSPLIT, 0)
_gather_b = _make_gather(LP - _PSPLIT, _PSPLIT)


_BB = 4096
_NQ = 12


def _quad_body(e_ref, w_ref, o_ref):
    eq = jnp.concatenate([e_ref[0], e_ref[1]], axis=1)
    res = lax.dot_general(
        w_ref[...], eq,
        dimension_numbers=(((1,), (1,)), ((), ())),
        preferred_element_type=jnp.float32)
    for r in range(4):
        o_ref[r] = res[DOUT * r:DOUT * (r + 1)]


def _quad_body_aliased(o_prev_ref, e_ref, w_ref, o_ref):
    del o_prev_ref
    _quad_body(e_ref, w_ref, o_ref)


def _project_quads_a(e2, w4):
    return pl.pallas_call(
        _quad_body,
        grid=(_PSPLIT // 2, B // _BB),
        in_specs=[
            pl.BlockSpec((2, _BB, 2 * DIM), lambda g, b: (g, b, 0)),
            pl.BlockSpec((4 * DOUT, 4 * DIM), lambda g, b: (0, 0)),
        ],
        out_specs=pl.BlockSpec((4, DOUT, _BB), lambda g, b: (g, 0, b)),
        out_shape=jax.ShapeDtypeStruct((L, DOUT, B), jnp.float32),
    )(e2, w4)


def _project_quads_b(out_prev, e2, w4):
    goff = _PSPLIT // 2
    return pl.pallas_call(
        _quad_body_aliased,
        grid=(_PSPLIT // 2, B // _BB),
        in_specs=[
            pl.BlockSpec(memory_space=pl.ANY),
            pl.BlockSpec((2, _BB, 2 * DIM), lambda g, b: (g, b, 0)),
            pl.BlockSpec((4 * DOUT, 4 * DIM), lambda g, b: (0, 0)),
        ],
        out_specs=pl.BlockSpec((4, DOUT, _BB), lambda g, b: (g + goff, 0, b)),
        out_shape=jax.ShapeDtypeStruct((L, DOUT, B), jnp.float32),
        input_output_aliases={0: 0},
    )(out_prev, e2, w4)


def _pair_body(o_prev_ref, e_ref, w_ref, o_ref):
    del o_prev_ref
    res = lax.dot_general(
        w_ref[...], e_ref[0],
        dimension_numbers=(((1,), (1,)), ((), ())),
        preferred_element_type=jnp.float32)
    o_ref[0] = res[:DOUT]
    o_ref[1] = res[DOUT:]


def _project_last_pair(out_prev, e2, w2):
    nb = LP - _PSPLIT
    return pl.pallas_call(
        _pair_body,
        grid=(B // _BB,),
        in_specs=[
            pl.BlockSpec(memory_space=pl.ANY),
            pl.BlockSpec((1, _BB, 2 * DIM), lambda b: (nb - 1, b, 0)),
            pl.BlockSpec((2 * DOUT, 2 * DIM), lambda b: (0, 0)),
        ],
        out_specs=pl.BlockSpec((2, DOUT, _BB), lambda b: (LP - 1, 0, b)),
        out_shape=jax.ShapeDtypeStruct((L, DOUT, B), jnp.float32),
        input_output_aliases={0: 0},
    )(out_prev, e2, w2)


def kernel(x, table, linear_layers_in):
    idx = jnp.pad(x.astype(jnp.int32), ((0, 0), (0, 128 - L))).reshape(B * 128)
    emb_a = _gather_a(table, idx)
    emb_b = _gather_b(table, idx)
    e2a = emb_a.reshape(_PSPLIT, B, 2 * DIM)
    e2b = emb_b.reshape(LP - _PSPLIT, B, 2 * DIM)
    w_t = jnp.transpose(linear_layers_in, (0, 2, 1)).reshape(DOUT, DIM)
    w4 = jnp.einsum('gh,ce->gche', jnp.eye(4, dtype=jnp.float32),
                    w_t).reshape(4 * DOUT, 4 * DIM)
    w2 = w4[:2 * DOUT, :2 * DIM]
    out_q = _project_quads_a(e2a, w4)
    out_q = _project_quads_b(out_q, e2b, w4)
    out_phys = _project_last_pair(out_q, e2b, w2)
    return out_phys.reshape(L, N_CTX, DIM, B).transpose(3, 0, 1, 2)
